# Initial kernel scaffold; baseline (speedup 1.0000x reference)
import jax
import jax.numpy as jnp
from jax.experimental import pallas as pl


def kernel(anchor_real, anchor_imag, positive_real, positive_imag, negative_real, negative_imag):
    raise NotImplementedError("write your pallas kernel here")


# speedup vs baseline: 1.0470x; 1.0470x over previous
import jax
import jax.numpy as jnp
from jax.experimental import pallas as pl
from jax.experimental.pallas import tpu as pltpu

N, D = 16384, 1024
BN = 512
TEMP = 0.1
MARGIN = 1.0


def _body(ar, ai, pr, pi, nr, ni, out_ref):
    step = pl.program_id(0)
    a_r = ar[...]
    a_i = ai[...]
    p_r = pr[...]
    p_i = pi[...]
    n_r = nr[...]
    n_i = ni[...]

    dot_p = jnp.sum(a_r * p_r + a_i * p_i, axis=1)
    dot_n = jnp.sum(a_r * n_r + a_i * n_i, axis=1)
    ssa = jnp.sum(a_r * a_r + a_i * a_i, axis=1)
    ssp = jnp.sum(p_r * p_r + p_i * p_i, axis=1)
    ssn = jnp.sum(n_r * n_r + n_i * n_i, axis=1)

    mag_a = jnp.sqrt(ssa + 1e-8)
    mag_p = jnp.sqrt(ssp + 1e-8)
    mag_n = jnp.sqrt(ssn + 1e-8)
    pos = dot_p / (mag_a * mag_p + 1e-8)
    neg = dot_n / (mag_a * mag_n + 1e-8)

    x = (neg - pos) / TEMP
    softplus = jnp.maximum(x, 0.0) + jnp.log1p(jnp.exp(-jnp.abs(x)))
    sep = jnp.maximum(neg + MARGIN, 0.0)
    part = jnp.sum(softplus + sep) * (1.0 / N)

    @pl.when(step == 0)
    def _():
        out_ref[0, 0] = 0.0

    out_ref[0, 0] += part


def kernel(anchor_real, anchor_imag, positive_real, positive_imag, negative_real, negative_imag):
    in_spec = pl.BlockSpec((BN, D), lambda i: (i, 0))
    out = pl.pallas_call(
        _body,
        grid=(N // BN,),
        in_specs=[in_spec] * 6,
        out_specs=pl.BlockSpec(memory_space=pltpu.SMEM),
        out_shape=jax.ShapeDtypeStruct((1, 1), jnp.float32),
        compiler_params=pltpu.CompilerParams(
            dimension_semantics=("arbitrary",),
        ),
    )(anchor_real, anchor_imag, positive_real, positive_imag, negative_real, negative_imag)
    return out[0, 0]
